# Initial kernel scaffold; baseline (speedup 1.0000x reference)
import jax
import jax.numpy as jnp
from jax.experimental import pallas as pl


def kernel(x, hashed_weight, weight_idx):
    raise NotImplementedError("write your pallas kernel here")


# speedup vs baseline: 107.3655x; 107.3655x over previous
import functools

import jax
import jax.numpy as jnp
from jax import lax
from jax.experimental import pallas as pl
from jax.experimental.pallas import tpu as pltpu
from jax.experimental.pallas import tpu_sc as plsc

NUM_EMB = 100000
D = 32
HW_SIZE = 320000
B = 16384
L = 50

NC = 2
NS = 16
NW = NC * NS

_MESH = plsc.VectorSubcoreMesh(core_axis_name="c", subcore_axis_name="s")
_PARAMS = pltpu.CompilerParams(use_tc_tiling_on_sc=False)

TBL_WORDS = NUM_EMB * D
A_PER_W = TBL_WORDS // NW
A_CH = 10000
A_NCH = A_PER_W // A_CH


@functools.partial(
    pl.kernel,
    out_type=jax.ShapeDtypeStruct((TBL_WORDS,), jnp.float32),
    mesh=_MESH,
    scratch_types=[
        pltpu.VMEM((A_CH,), jnp.int32),
        pltpu.VMEM((A_CH,), jnp.float32),
        pltpu.SemaphoreType.DMA,
    ],
    compiler_params=_PARAMS,
)
def _build_table(hw_hbm, widx_hbm, tbl_hbm, idx_v, val_v, sem):
    wid = lax.axis_index("s") * NC + lax.axis_index("c")
    base = wid * A_PER_W
    for k in range(A_NCH):
        off = base + k * A_CH
        pltpu.sync_copy(widx_hbm.at[pl.ds(off, A_CH)], idx_v)
        pltpu.async_copy(hw_hbm.at[idx_v], val_v, sem).wait()
        pltpu.sync_copy(val_v, tbl_hbm.at[pl.ds(off, A_CH)])


BAGS_PER_W = B // NW
G = 32
B_NCH = BAGS_PER_W // G


@functools.partial(
    pl.kernel,
    out_type=jax.ShapeDtypeStruct((B, D), jnp.float32),
    mesh=_MESH,
    scratch_types=[
        pltpu.VMEM((G * L,), jnp.int32),
        pltpu.VMEM((G * L, D), jnp.float32),
        pltpu.VMEM((G, D), jnp.float32),
        pltpu.SemaphoreType.DMA,
    ],
    compiler_params=_PARAMS,
)
def _bag_sum(tbl_hbm, x_hbm, out_hbm, xidx_v, rows_v, ob_v, sem):
    wid = lax.axis_index("s") * NC + lax.axis_index("c")
    bag0 = wid * BAGS_PER_W
    for k in range(B_NCH):
        bb = bag0 + k * G
        pltpu.sync_copy(x_hbm.at[pl.ds(bb * L, G * L)], xidx_v)
        pltpu.async_copy(tbl_hbm.at[xidx_v], rows_v, sem).wait()

        def body(b, _):
            r0 = b * L
            acc0 = jnp.zeros((16,), jnp.float32)
            acc1 = jnp.zeros((16,), jnp.float32)
            for l in range(L):
                acc0 = acc0 + rows_v[r0 + l, pl.ds(0, 16)]
                acc1 = acc1 + rows_v[r0 + l, pl.ds(16, 16)]
            ob_v[b, pl.ds(0, 16)] = acc0
            ob_v[b, pl.ds(16, 16)] = acc1
            return 0

        lax.fori_loop(0, G, body, 0)
        pltpu.sync_copy(ob_v, out_hbm.at[pl.ds(bb, G)])


def kernel(x, hashed_weight, weight_idx):
    tbl_flat = _build_table(hashed_weight, weight_idx.reshape(-1))
    tbl = tbl_flat.reshape(NUM_EMB, D)
    return _bag_sum(tbl, x.reshape(-1))


# speedup vs baseline: 160.5171x; 1.4951x over previous
import functools

import jax
import jax.numpy as jnp
import numpy as np
from jax import lax
from jax.experimental import pallas as pl
from jax.experimental.pallas import tpu as pltpu
from jax.experimental.pallas import tpu_sc as plsc

NUM_EMB = 100000
D = 32
HW_SIZE = 320000
B = 16384
L = 50

NC = 2
NS = 16
NW = NC * NS

_MESH = plsc.VectorSubcoreMesh(core_axis_name="c", subcore_axis_name="s")
_PARAMS = pltpu.CompilerParams(use_tc_tiling_on_sc=False)

TBL_WORDS = NUM_EMB * D
A_PER_W = TBL_WORDS // NW
A_CH = 10000
A_NCH = A_PER_W // A_CH
HW_PER_T = HW_SIZE // NS


@functools.partial(
    pl.kernel,
    out_type=jax.ShapeDtypeStruct((TBL_WORDS,), jnp.float32),
    mesh=_MESH,
    scratch_types=[
        pltpu.VMEM_SHARED((HW_SIZE,), jnp.float32),
        [pltpu.VMEM((A_CH,), jnp.int32)] * 2,
        [pltpu.VMEM((A_CH,), jnp.float32)] * 2,
        [pltpu.SemaphoreType.DMA] * 2,
    ],
    compiler_params=_PARAMS,
)
def _build_table(hw_hbm, widx_hbm, tbl_hbm, hw_sh, idx_v, val_v, sem):
    s = lax.axis_index("s")
    wid = s * NC + lax.axis_index("c")
    for j in range(2):
        off = s * HW_PER_T + j * A_CH
        pltpu.sync_copy(hw_hbm.at[pl.ds(off, A_CH)], val_v[j])
        pltpu.sync_copy(val_v[j], hw_sh.at[pl.ds(off, A_CH)])
    plsc.subcore_barrier()

    base = wid * A_PER_W
    pltpu.sync_copy(widx_hbm.at[pl.ds(base, A_CH)], idx_v[0])
    cps = [pltpu.async_copy(hw_sh.at[idx_v[0]], val_v[0], sem[0])]
    for k in range(1, A_NCH + 1):
        if k < A_NCH:
            pltpu.sync_copy(widx_hbm.at[pl.ds(base + k * A_CH, A_CH)],
                            idx_v[k % 2])
            cps.append(pltpu.async_copy(hw_sh.at[idx_v[k % 2]],
                                        val_v[k % 2], sem[k % 2]))
        cps[k - 1].wait()
        pltpu.sync_copy(val_v[(k - 1) % 2],
                        tbl_hbm.at[pl.ds(base + (k - 1) * A_CH, A_CH)])


BAGS_PER_W = B // NW
BAGS_PER_SC = B // NC
G = 32
GL = G * L
B_NCH = BAGS_PER_W // G


@functools.partial(
    pl.kernel,
    out_type=jax.ShapeDtypeStruct((B, D), jnp.float32),
    mesh=_MESH,
    scratch_types=[
        pltpu.VMEM_SHARED((BAGS_PER_SC, D), jnp.float32),
        [pltpu.VMEM((GL,), jnp.int32)] * 2,
        [pltpu.VMEM((GL, D), jnp.float32)] * 2,
        pltpu.VMEM((GL,), jnp.int32),
        [pltpu.SemaphoreType.DMA] * 2,
    ],
    compiler_params=_PARAMS,
)
def _bag_sum(tbl_hbm, x_hbm, sidx_hbm, out_hbm, acc_sh, xidx_v, rows_v,
             sidx_v, sem):
    s = lax.axis_index("s")
    wid = s * NC + lax.axis_index("c")
    bag0 = wid * BAGS_PER_W
    accrow0 = s * BAGS_PER_W

    def zbody(i, _):
        z = jnp.zeros((16,), jnp.float32)
        rows_v[0][i, pl.ds(0, 16)] = z
        rows_v[0][i, pl.ds(16, 16)] = z
        return 0
    lax.fori_loop(0, BAGS_PER_W, zbody, 0)
    pltpu.sync_copy(rows_v[0].at[pl.ds(0, BAGS_PER_W)],
                    acc_sh.at[pl.ds(accrow0, BAGS_PER_W)])

    def start(k):
        pltpu.sync_copy(x_hbm.at[pl.ds((bag0 + k * G) * L, GL)],
                        xidx_v[k % 2])
        return pltpu.async_copy(tbl_hbm.at[xidx_v[k % 2]],
                                rows_v[k % 2], sem[k % 2])

    cps = [start(0)]
    for k in range(1, B_NCH + 1):
        if k < B_NCH:
            cps.append(start(k))
        kk = k - 1
        pltpu.sync_copy(sidx_hbm.at[pl.ds((s * B_NCH + kk) * GL, GL)], sidx_v)
        cps[kk].wait()
        pltpu.sync_copy(rows_v[kk % 2], acc_sh.at[sidx_v], add=True)

    pltpu.sync_copy(acc_sh.at[pl.ds(accrow0, BAGS_PER_W)],
                    rows_v[0].at[pl.ds(0, BAGS_PER_W)])
    pltpu.sync_copy(rows_v[0].at[pl.ds(0, BAGS_PER_W)],
                    out_hbm.at[pl.ds(bag0, BAGS_PER_W)])


_SIDX = jnp.asarray(
    (np.arange(NS, dtype=np.int32)[:, None, None] * BAGS_PER_W
     + np.arange(B_NCH, dtype=np.int32)[None, :, None] * G
     + np.arange(GL, dtype=np.int32)[None, None, :] // L)
    .reshape(-1))


def kernel(x, hashed_weight, weight_idx):
    tbl_flat = _build_table(hashed_weight, weight_idx.reshape(-1))
    tbl = tbl_flat.reshape(NUM_EMB, D)
    return _bag_sum(tbl, x.reshape(-1), _SIDX)


# speedup vs baseline: 162.0265x; 1.0094x over previous
import functools

import jax
import jax.numpy as jnp
import numpy as np
from jax import lax
from jax.experimental import pallas as pl
from jax.experimental.pallas import tpu as pltpu
from jax.experimental.pallas import tpu_sc as plsc

NUM_EMB = 100000
D = 32
HW_SIZE = 320000
B = 16384
L = 50

NC = 2
NS = 16
NW = NC * NS

_MESH = plsc.VectorSubcoreMesh(core_axis_name="c", subcore_axis_name="s")
_PARAMS = pltpu.CompilerParams(use_tc_tiling_on_sc=False)

TBL_WORDS = NUM_EMB * D
A_PER_W = TBL_WORDS // NW
A_CH = 10000
A_NCH = A_PER_W // A_CH
HW_PER_T = HW_SIZE // NS


@functools.partial(
    pl.kernel,
    out_type=jax.ShapeDtypeStruct((TBL_WORDS,), jnp.float32),
    mesh=_MESH,
    scratch_types=[
        pltpu.VMEM_SHARED((HW_SIZE,), jnp.float32),
        [pltpu.VMEM((A_CH,), jnp.int32)] * 2,
        [pltpu.VMEM((A_CH,), jnp.float32)] * 2,
        [pltpu.SemaphoreType.DMA] * 2,
    ],
    compiler_params=_PARAMS,
)
def _build_table(hw_hbm, widx_hbm, tbl_hbm, hw_sh, idx_v, val_v, sem):
    s = lax.axis_index("s")
    wid = s * NC + lax.axis_index("c")
    for j in range(2):
        off = s * HW_PER_T + j * A_CH
        pltpu.sync_copy(hw_hbm.at[pl.ds(off, A_CH)], val_v[j])
        pltpu.sync_copy(val_v[j], hw_sh.at[pl.ds(off, A_CH)])
    plsc.subcore_barrier()

    base = wid * A_PER_W
    pltpu.sync_copy(widx_hbm.at[pl.ds(base, A_CH)], idx_v[0])
    cps = [pltpu.async_copy(hw_sh.at[idx_v[0]], val_v[0], sem[0])]
    for k in range(1, A_NCH + 1):
        if k < A_NCH:
            pltpu.sync_copy(widx_hbm.at[pl.ds(base + k * A_CH, A_CH)],
                            idx_v[k % 2])
            cps.append(pltpu.async_copy(hw_sh.at[idx_v[k % 2]],
                                        val_v[k % 2], sem[k % 2]))
        cps[k - 1].wait()
        pltpu.sync_copy(val_v[(k - 1) % 2],
                        tbl_hbm.at[pl.ds(base + (k - 1) * A_CH, A_CH)])


BAGS_PER_W = B // NW
BAGS_PER_SC = B // NC
G = 32
GL = G * L
B_NCH = BAGS_PER_W // G


@functools.partial(
    pl.kernel,
    out_type=jax.ShapeDtypeStruct((B, D), jnp.float32),
    mesh=_MESH,
    scratch_types=[
        pltpu.VMEM_SHARED((BAGS_PER_SC, D), jnp.float32),
        [pltpu.VMEM((GL,), jnp.int32)] * 2,
        [pltpu.VMEM((GL, D), jnp.float32)] * 2,
        [pltpu.VMEM((GL,), jnp.int32)] * 2,
        [pltpu.SemaphoreType.DMA] * 2,
        [pltpu.SemaphoreType.DMA] * 2,
    ],
    compiler_params=_PARAMS,
)
def _bag_sum(tbl_hbm, x_hbm, sidx_hbm, out_hbm, acc_sh, xidx_v, rows_v,
             sidx_v, gsem, ssem):
    s = lax.axis_index("s")
    wid = s * NC + lax.axis_index("c")
    bag0 = wid * BAGS_PER_W
    accrow0 = s * BAGS_PER_W

    def start(k):
        pltpu.sync_copy(x_hbm.at[pl.ds((bag0 + k * G) * L, GL)],
                        xidx_v[k % 2])
        return pltpu.async_copy(tbl_hbm.at[xidx_v[k % 2]],
                                rows_v[k % 2], gsem[k % 2])

    cps = [start(0)]

    def zbody(i, _):
        z = jnp.zeros((16,), jnp.float32)
        rows_v[1][i, pl.ds(0, 16)] = z
        rows_v[1][i, pl.ds(16, 16)] = z
        return 0
    lax.fori_loop(0, BAGS_PER_W, zbody, 0)
    pltpu.sync_copy(rows_v[1].at[pl.ds(0, BAGS_PER_W)],
                    acc_sh.at[pl.ds(accrow0, BAGS_PER_W)])

    scs = []
    for k in range(1, B_NCH + 1):
        if k < B_NCH:
            if k >= 2:
                scs[k - 2].wait()
            cps.append(start(k))
        kk = k - 1
        pltpu.sync_copy(sidx_hbm.at[pl.ds((s * B_NCH + kk) * GL, GL)],
                        sidx_v[kk % 2])
        cps[kk].wait()
        scs.append(pltpu.async_copy(rows_v[kk % 2],
                                    acc_sh.at[sidx_v[kk % 2]],
                                    ssem[kk % 2], add=True))

    scs[B_NCH - 2].wait()
    scs[B_NCH - 1].wait()
    pltpu.sync_copy(acc_sh.at[pl.ds(accrow0, BAGS_PER_W)],
                    rows_v[0].at[pl.ds(0, BAGS_PER_W)])
    pltpu.sync_copy(rows_v[0].at[pl.ds(0, BAGS_PER_W)],
                    out_hbm.at[pl.ds(bag0, BAGS_PER_W)])


_SIDX = jnp.asarray(
    (np.arange(NS, dtype=np.int32)[:, None, None] * BAGS_PER_W
     + np.arange(B_NCH, dtype=np.int32)[None, :, None] * G
     + np.arange(GL, dtype=np.int32)[None, None, :] // L)
    .reshape(-1))


def kernel(x, hashed_weight, weight_idx):
    tbl_flat = _build_table(hashed_weight, weight_idx.reshape(-1))
    tbl = tbl_flat.reshape(NUM_EMB, D)
    return _bag_sum(tbl, x.reshape(-1), _SIDX)
